# baseline (speedup 1.0000x reference)
import functools
import math

import jax
import jax.numpy as jnp
from jax.experimental import pallas as pl
from jax.experimental.pallas import tpu as pltpu

_ALPHA = 0.5
_N_HEAD = 16
_INV_SQRT2 = 1.0 / math.sqrt(2.0)


def _ncn_kernel(x_ref, xa_ref, w_ref, yi_ref, ya_ref):
    b = pl.program_id(0)
    x = x_ref[b]
    cache = xa_ref[0]
    w_read = w_ref[0:1, :]
    w_write = w_ref[1:2, :]

    S, D = x.shape
    C = cache.shape[0]
    H = _N_HEAD
    Dh = D // H

    q = x * w_read
    qb = q.astype(jnp.bfloat16)
    kb = cache.astype(jnp.bfloat16)

    shs = []
    for h in range(H):
        qh = qb[:, h * Dh:(h + 1) * Dh]
        kh = kb[:, h * Dh:(h + 1) * Dh]
        shs.append(jax.lax.dot_general(
            qh, kh, (((1,), (1,)), ((), ())),
            preferred_element_type=jnp.float32))

    sums = list(shs)
    while len(sums) > 1:
        sums = [a + b for a, b in zip(sums[::2], sums[1::2])]
    tok_scores = sums[0]
    rows_vec = jnp.argmax(tok_scores, axis=1).astype(jnp.int32)
    rows = [rows_vec[s] for s in range(S)]

    s_all = jnp.concatenate(shs, axis=0)
    m = jnp.max(s_all, axis=1, keepdims=True)
    e = jnp.exp(s_all - m)
    attn = (e / jnp.sum(e, axis=1, keepdims=True)).astype(jnp.bfloat16)

    read_parts = []
    for h in range(H):
        kh = kb[:, h * Dh:(h + 1) * Dh]
        read_parts.append(jax.lax.dot_general(
            attn[h * S:(h + 1) * S, :], kh, (((1,), (0,)), ((), ())),
            preferred_element_type=jnp.float32))

    read = jnp.concatenate(read_parts, axis=1)
    pre = (x + _ALPHA * read) * _INV_SQRT2
    yi_ref[b] = jnp.maximum(pre, 0.0)

    write_val = (1.0 - _ALPHA) * (x * w_write)
    ya_ref[0] = _ALPHA * cache
    for s in range(S):
        ya_ref[0, pl.ds(rows[s], 1), :] += write_val[s:s + 1, :]


@functools.partial(jax.jit, static_argnames=())
def kernel(x, xa, W):
    B, S, D = x.shape
    C = xa.shape[1]
    Dh = D // _N_HEAD
    w2 = W.reshape(2, D) * jnp.array(
        [[1.0 / math.sqrt(Dh)], [1.0]], dtype=jnp.float32)
    grid = (B,)
    yi, ya = pl.pallas_call(
        _ncn_kernel,
        grid=grid,
        in_specs=[
            pl.BlockSpec((B, S, D), lambda b: (0, 0, 0)),
            pl.BlockSpec((1, C, D), lambda b: (b, 0, 0)),
            pl.BlockSpec((2, D), lambda b: (0, 0)),
        ],
        out_specs=[
            pl.BlockSpec((B, S, D), lambda b: (0, 0, 0)),
            pl.BlockSpec((1, C, D), lambda b: (b, 0, 0)),
        ],
        out_shape=[
            jax.ShapeDtypeStruct((B, S, D), jnp.float32),
            jax.ShapeDtypeStruct((B, C, D), jnp.float32),
        ],
        compiler_params=pltpu.CompilerParams(
            dimension_semantics=("arbitrary",)),
    )(x, xa, w2)
    return yi, ya


# speedup vs baseline: 1.0013x; 1.0013x over previous
import functools
import math

import jax
import jax.numpy as jnp
from jax.experimental import pallas as pl

_ALPHA = 0.5
_N_HEAD = 16
_INV_SQRT2 = 1.0 / math.sqrt(2.0)


def _ncn_kernel(x_ref, xa_ref, w_ref, yi_ref, ya_ref):
    b = pl.program_id(0)
    x = x_ref[b]
    cache = xa_ref[0]
    w_read = w_ref[0:1, :]
    w_write = w_ref[1:2, :]

    S, D = x.shape
    C = cache.shape[0]
    H = _N_HEAD
    Dh = D // H

    q = x * w_read
    qb = q.astype(jnp.bfloat16)
    kb = cache.astype(jnp.bfloat16)

    ya_ref[0] = _ALPHA * cache

    shs = []
    for h in range(H):
        qh = qb[:, h * Dh:(h + 1) * Dh]
        kh = kb[:, h * Dh:(h + 1) * Dh]
        shs.append(jax.lax.dot_general(
            qh, kh, (((1,), (1,)), ((), ())),
            preferred_element_type=jnp.float32))

    sums = list(shs)
    while len(sums) > 1:
        sums = [a + b for a, b in zip(sums[::2], sums[1::2])]
    tok_scores = sums[0]
    rows_vec = jnp.argmax(tok_scores, axis=1).astype(jnp.int32)
    rows = [rows_vec[s] for s in range(S)]

    s_all = jnp.concatenate(shs, axis=0)
    m = jnp.max(s_all, axis=1, keepdims=True)
    e = jnp.exp(s_all - m)
    attn = (e / jnp.sum(e, axis=1, keepdims=True)).astype(jnp.bfloat16)

    read_parts = []
    for h in range(H):
        kh = kb[:, h * Dh:(h + 1) * Dh]
        read_parts.append(jax.lax.dot_general(
            attn[h * S:(h + 1) * S, :], kh, (((1,), (0,)), ((), ())),
            preferred_element_type=jnp.float32))

    read = jnp.concatenate(read_parts, axis=1)
    pre = (x + _ALPHA * read) * _INV_SQRT2
    yi_ref[b] = jnp.maximum(pre, 0.0)

    write_val = (1.0 - _ALPHA) * (x * w_write)
    for s in range(S):
        ya_ref[0, pl.ds(rows[s], 1), :] += write_val[s:s + 1, :]


@functools.partial(jax.jit, static_argnames=())
def kernel(x, xa, W):
    B, S, D = x.shape
    C = xa.shape[1]
    Dh = D // _N_HEAD
    w2 = W.reshape(2, D) * jnp.array(
        [[1.0 / math.sqrt(Dh)], [1.0]], dtype=jnp.float32)
    grid = (B,)
    yi, ya = pl.pallas_call(
        _ncn_kernel,
        grid=grid,
        in_specs=[
            pl.BlockSpec((B, S, D), lambda b: (0, 0, 0)),
            pl.BlockSpec((1, C, D), lambda b: (b, 0, 0)),
            pl.BlockSpec((2, D), lambda b: (0, 0)),
        ],
        out_specs=[
            pl.BlockSpec((B, S, D), lambda b: (0, 0, 0)),
            pl.BlockSpec((1, C, D), lambda b: (b, 0, 0)),
        ],
        out_shape=[
            jax.ShapeDtypeStruct((B, S, D), jnp.float32),
            jax.ShapeDtypeStruct((B, C, D), jnp.float32),
        ],
    )(x, xa, w2)
    return yi, ya
